# baseline (speedup 1.0000x reference)
import functools

import jax
import jax.numpy as jnp
from jax import lax
from jax.experimental import pallas as pl
from jax.experimental.pallas import tpu as pltpu
import jax.experimental.pallas.tpu_sc as plsc

N = 10000
E = 320000
NC, NS, L = 2, 16, 16
CH = 128
CPB = 8
NB0 = 15
NB1 = 5
NBMAX = max(NB0, NB1)
K0 = NB0 * CPB
K1 = NB1 * CPB
NCHUNK = NS * (K0 + K1)
EPAD = NCHUNK * CH
N_SH = NS * 640
BR = 1000


def _edge_agg_body(D, y_hbm, src_hbm, dst_hbm, out_hbm, srcw_v, dstw_v,
                   rows_v, z_v, shared, *sems):
  c = lax.axis_index("c")
  s = lax.axis_index("s")

  base = jnp.where(c == 0, s * K0, NS * K0 + s * K1)

  @pl.when(c == 0)
  def _():
    pltpu.sync_copy(src_hbm.at[pl.ds(base, K0)], srcw_v.at[pl.ds(0, K0)])
    pltpu.sync_copy(dst_hbm.at[pl.ds(base, K0)], dstw_v.at[pl.ds(0, K0)])

  @pl.when(c != 0)
  def _():
    pltpu.sync_copy(src_hbm.at[pl.ds(base, K1)], srcw_v.at[pl.ds(0, K1)])
    pltpu.sync_copy(dst_hbm.at[pl.ds(base, K1)], dstw_v.at[pl.ds(0, K1)])

  for b in range(CPB):
    pltpu.async_copy(y_hbm.at[srcw_v.at[b]], rows_v.at[b], sems[b])

  def zrow(i, _):
    for j in range(D // L):
      z_v[i, pl.ds(j * L, L)] = jnp.zeros((L,), jnp.float32)
    return 0
  lax.fori_loop(0, CH, zrow, 0, unroll=4)
  r0 = s * 640
  def icopy(k, _):
    pltpu.sync_copy(z_v, shared.at[pl.ds(r0 + k * CH, CH)])
    return 0
  lax.fori_loop(0, 640 // CH, icopy, 0)
  plsc.subcore_barrier()

  def gwait(b):
    pltpu.make_async_copy(y_hbm.at[srcw_v.at[0]], rows_v.at[b],
                          sems[b]).wait()

  def swait(b):
    pltpu.make_async_copy(rows_v.at[b], shared.at[dstw_v.at[0]],
                          sems[b]).wait()

  nb = jnp.where(c == 0, NB0, NB1)
  def pbody(g, _):
    for b in range(CPB):
      gwait(b)
      pltpu.async_copy(rows_v.at[b], shared.at[dstw_v.at[g * CPB + b]],
                       sems[b], add=True)
    for b in range(CPB):
      swait(b)
      pltpu.async_copy(y_hbm.at[srcw_v.at[(g + 1) * CPB + b]],
                       rows_v.at[b], sems[b])
    return 0
  lax.fori_loop(0, nb - 1, pbody, 0)
  for b in range(CPB):
    gwait(b)
    pltpu.async_copy(rows_v.at[b], shared.at[dstw_v.at[(nb - 1) * CPB + b]],
                     sems[b], add=True)
  for b in range(CPB):
    swait(b)
  plsc.subcore_barrier()

  for k in range(640 // CH):
    pltpu.sync_copy(shared.at[pl.ds(r0 + k * CH, CH)], rows_v.at[0])
    pltpu.sync_copy(rows_v.at[0], out_hbm.at[c, pl.ds(r0 + k * CH, CH)])


def _make_edge_agg(D):
  mesh = plsc.VectorSubcoreMesh(core_axis_name="c", subcore_axis_name="s",
                                num_cores=NC, num_subcores=NS)
  scratch = [
      pltpu.VMEM((NBMAX * CPB, CH), jnp.int32),
      pltpu.VMEM((NBMAX * CPB, CH), jnp.int32),
      pltpu.VMEM((CPB, CH, D), jnp.float32),
      pltpu.VMEM((CH, D), jnp.float32),
      pltpu.VMEM_SHARED((N_SH, D), jnp.float32),
  ] + [pltpu.SemaphoreType.DMA] * CPB
  return pl.kernel(
      functools.partial(_edge_agg_body, D),
      out_type=jax.ShapeDtypeStruct((NC, N_SH, D), jnp.float32),
      mesh=mesh, scratch_types=scratch,
      compiler_params=pltpu.CompilerParams(use_tc_tiling_on_sc=False),
      name=f"edge_agg_d{D}")


def _mm(x, w):
  return lax.dot_general(x, w, (((1,), (1,)), ((), ())),
                         preferred_element_type=jnp.float32)


def _tc_a_body(x_ref, wl_ref, wr_ref, bl_ref, y_ref, r_ref):
  xb = x_ref[...]
  y = _mm(xb, wl_ref[...])
  aug = jnp.where(lax.broadcasted_iota(jnp.int32, (BR, L), 1) == 0, 1.0, 0.0)
  y_ref[...] = jnp.concatenate([y, aug], axis=1)
  r_ref[...] = _mm(xb, wr_ref[...]) + bl_ref[...]


def _tc_mid_body(p_ref, r_ref, wl_ref, wr_ref, bl_ref,
                 y_ref, r2_ref, ci_ref):
  agg = p_ref[0, :, 0:32] + p_ref[1, :, 0:32]
  cnt = p_ref[0, :, 32:33] + p_ref[1, :, 32:33]
  cinv = 1.0 / jnp.maximum(cnt, 1.0)
  h = jax.nn.relu(agg * cinv + r_ref[...])
  y_ref[...] = _mm(h, wl_ref[...])
  r2_ref[...] = _mm(h, wr_ref[...]) + bl_ref[...]
  ci_ref[...] = jnp.broadcast_to(cinv, (BR, L))


def _tc_out_body(p_ref, ci_ref, r_ref, wfc_ref, bfc_ref, o_ref):
  agg = p_ref[0] + p_ref[1]
  h = jax.nn.relu(agg * ci_ref[...] + r_ref[...])
  o_ref[...] = _mm(h, wfc_ref[...]) + bfc_ref[...]


def _row_spec(d):
  return pl.BlockSpec((BR, d), lambda i: (i, 0))


def _part_spec(d):
  return pl.BlockSpec((NC, BR, d), lambda i: (0, i, 0))


def _full_spec(a, b):
  return pl.BlockSpec((a, b), lambda i: (0, 0))


def kernel(x, edge_index, Wl1, bl1, Wr1, Wl2, bl2, Wr2, Wfc, bfc):
  ei = edge_index.astype(jnp.int32)
  pad = EPAD - E
  src = jnp.concatenate([ei[0], jnp.zeros((pad,), jnp.int32)])
  src = src.reshape(NCHUNK, CH)
  pad_dst = N + jnp.arange(pad, dtype=jnp.int32) % (N_SH - N)
  dst = jnp.concatenate([ei[1], pad_dst])
  dst = dst.reshape(NCHUNK, CH)

  grid = N // BR

  y1, r1 = pl.pallas_call(
      _tc_a_body,
      grid=(grid,),
      in_specs=[_row_spec(128), _full_spec(32, 128), _full_spec(32, 128),
                _full_spec(1, 32)],
      out_specs=[_row_spec(48), _row_spec(32)],
      out_shape=[jax.ShapeDtypeStruct((N, 48), jnp.float32),
                 jax.ShapeDtypeStruct((N, 32), jnp.float32)],
  )(x, Wl1, Wr1, bl1.reshape(1, 32))

  p1 = _make_edge_agg(48)(y1, src, dst)

  y2, r2, cinv = pl.pallas_call(
      _tc_mid_body,
      grid=(grid,),
      in_specs=[_part_spec(48), _row_spec(32),
                _full_spec(16, 32), _full_spec(16, 32), _full_spec(1, 16)],
      out_specs=[_row_spec(16), _row_spec(16), _row_spec(L)],
      out_shape=[jax.ShapeDtypeStruct((N, 16), jnp.float32)] * 2
      + [jax.ShapeDtypeStruct((N, L), jnp.float32)],
  )(p1, r1, Wl2, Wr2, bl2.reshape(1, 16))

  p2 = _make_edge_agg(16)(y2, src, dst)

  out = pl.pallas_call(
      _tc_out_body,
      grid=(grid,),
      in_specs=[_part_spec(16), _row_spec(L), _row_spec(16),
                _full_spec(2, 16), _full_spec(1, 2)],
      out_specs=_row_spec(2),
      out_shape=jax.ShapeDtypeStruct((N, 2), jnp.float32),
  )(p2, cinv, r2, Wfc, bfc.reshape(1, 2))
  return out


# speedup vs baseline: 1.6109x; 1.6109x over previous
import functools

import jax
import jax.numpy as jnp
from jax import lax
from jax.experimental import pallas as pl
from jax.experimental.pallas import tpu as pltpu
import jax.experimental.pallas.tpu_sc as plsc

N = 10000
E = 320000
NC, NS, L = 2, 16, 16
CH = 128
CPB = 8
NB0 = 10
NB1 = 10
NBMAX = max(NB0, NB1)
K0 = NB0 * CPB
K1 = NB1 * CPB
NCHUNK = NS * (K0 + K1)
EPAD = NCHUNK * CH
N_SH = NS * 640
BR = 1000


def _edge_agg_body(with_cnt, D, y_hbm, src_hbm, dst_hbm, *refs):
  if with_cnt:
    (out_hbm, cnt_hbm, srcw_v, dstw_v, rows_v, z_v, z16_v, ones_v,
     table, shared, shared_cnt) = refs[:11]
  else:
    (out_hbm, srcw_v, dstw_v, rows_v, z_v, table, shared) = refs[:7]
  sems = refs[-CPB:]
  c = lax.axis_index("c")
  s = lax.axis_index("s")

  base = jnp.where(c == 0, s * K0, NS * K0 + s * K1)

  @pl.when(c == 0)
  def _():
    pltpu.sync_copy(src_hbm.at[pl.ds(base, K0)], srcw_v.at[pl.ds(0, K0)])
    pltpu.sync_copy(dst_hbm.at[pl.ds(base, K0)], dstw_v.at[pl.ds(0, K0)])

  @pl.when(c != 0)
  def _():
    pltpu.sync_copy(src_hbm.at[pl.ds(base, K1)], srcw_v.at[pl.ds(0, K1)])
    pltpu.sync_copy(dst_hbm.at[pl.ds(base, K1)], dstw_v.at[pl.ds(0, K1)])

  t0 = s * 625
  for k in range(5):
    pltpu.sync_copy(y_hbm.at[pl.ds(t0 + k * 125, 125)],
                    rows_v.at[0, pl.ds(0, 125)])
    pltpu.sync_copy(rows_v.at[0, pl.ds(0, 125)],
                    table.at[pl.ds(t0 + k * 125, 125)])

  def zrow(i, _):
    for j in range(D // L):
      z_v[i, pl.ds(j * L, L)] = jnp.zeros((L,), jnp.float32)
    if with_cnt:
      z16_v[i, pl.ds(0, L)] = jnp.zeros((L,), jnp.float32)
      ones_v[i, pl.ds(0, L)] = jnp.ones((L,), jnp.float32)
    return 0
  lax.fori_loop(0, CH, zrow, 0, unroll=4)
  r0 = s * 640
  def icopy(k, _):
    pltpu.sync_copy(z_v, shared.at[pl.ds(r0 + k * CH, CH)])
    if with_cnt:
      pltpu.sync_copy(z16_v, shared_cnt.at[pl.ds(r0 + k * CH, CH)])
    return 0
  lax.fori_loop(0, 640 // CH, icopy, 0)
  plsc.subcore_barrier()

  for b in range(CPB):
    pltpu.async_copy(table.at[srcw_v.at[b]], rows_v.at[b], sems[b])

  def gwait(b):
    pltpu.make_async_copy(table.at[srcw_v.at[0]], rows_v.at[b],
                          sems[b]).wait()

  def swait(b):
    pltpu.make_async_copy(rows_v.at[b], shared.at[dstw_v.at[0]],
                          sems[b]).wait()
    if with_cnt:
      pltpu.make_async_copy(ones_v, shared_cnt.at[dstw_v.at[0]],
                            sems[b]).wait()

  def fire_scatter(b, j):
    pltpu.async_copy(rows_v.at[b], shared.at[dstw_v.at[j]], sems[b],
                     add=True)
    if with_cnt:
      pltpu.async_copy(ones_v, shared_cnt.at[dstw_v.at[j]], sems[b],
                       add=True)

  nb = jnp.where(c == 0, NB0, NB1)
  def pbody(g, _):
    for b in range(CPB):
      gwait(b)
      fire_scatter(b, g * CPB + b)
    for b in range(CPB):
      swait(b)
      pltpu.async_copy(table.at[srcw_v.at[(g + 1) * CPB + b]],
                       rows_v.at[b], sems[b])
    return 0
  lax.fori_loop(0, nb - 1, pbody, 0)
  for b in range(CPB):
    gwait(b)
    fire_scatter(b, (nb - 1) * CPB + b)
  for b in range(CPB):
    swait(b)
  plsc.subcore_barrier()

  for k in range(640 // CH):
    pltpu.sync_copy(shared.at[pl.ds(r0 + k * CH, CH)], rows_v.at[0])
    pltpu.sync_copy(rows_v.at[0], out_hbm.at[c, pl.ds(r0 + k * CH, CH)])
  if with_cnt:
    for k in range(640 // CH):
      pltpu.sync_copy(shared_cnt.at[pl.ds(r0 + k * CH, CH)], ones_v)
      pltpu.sync_copy(ones_v, cnt_hbm.at[c, pl.ds(r0 + k * CH, CH)])


def _make_edge_agg(D, with_cnt):
  mesh = plsc.VectorSubcoreMesh(core_axis_name="c", subcore_axis_name="s",
                                num_cores=NC, num_subcores=NS)
  out_type = [jax.ShapeDtypeStruct((NC, N_SH, D), jnp.float32)]
  scratch = [
      pltpu.VMEM((NBMAX * CPB, CH), jnp.int32),
      pltpu.VMEM((NBMAX * CPB, CH), jnp.int32),
      pltpu.VMEM((CPB, CH, D), jnp.float32),
      pltpu.VMEM((CH, D), jnp.float32),
  ]
  if with_cnt:
    out_type.append(jax.ShapeDtypeStruct((NC, N_SH, L), jnp.float32))
    scratch.append(pltpu.VMEM((CH, L), jnp.float32))
    scratch.append(pltpu.VMEM((CH, L), jnp.float32))
  scratch.append(pltpu.VMEM_SHARED((N, D), jnp.float32))
  scratch.append(pltpu.VMEM_SHARED((N_SH, D), jnp.float32))
  if with_cnt:
    scratch.append(pltpu.VMEM_SHARED((N_SH, L), jnp.float32))
  scratch.extend([pltpu.SemaphoreType.DMA] * CPB)
  return pl.kernel(
      functools.partial(_edge_agg_body, with_cnt, D),
      out_type=out_type, mesh=mesh, scratch_types=scratch,
      compiler_params=pltpu.CompilerParams(use_tc_tiling_on_sc=False),
      name=f"edge_agg_d{D}")


def _mm(x, w):
  return lax.dot_general(x, w, (((1,), (1,)), ((), ())),
                         preferred_element_type=jnp.float32)


def _tc_a_body(x_ref, wl_ref, wr_ref, bl_ref, y_ref, r_ref):
  xb = x_ref[...]
  y_ref[...] = _mm(xb, wl_ref[...])
  r_ref[...] = _mm(xb, wr_ref[...]) + bl_ref[...]


def _tc_mid_body(p_ref, c_ref, r_ref, wl_ref, wr_ref, bl_ref,
                 y_ref, r2_ref, ci_ref):
  agg = p_ref[0] + p_ref[1]
  cnt = (c_ref[0] + c_ref[1])[:, 0:1]
  cinv = 1.0 / jnp.maximum(cnt, 1.0)
  h = jax.nn.relu(agg * cinv + r_ref[...])
  y_ref[...] = _mm(h, wl_ref[...])
  r2_ref[...] = _mm(h, wr_ref[...]) + bl_ref[...]
  ci_ref[...] = jnp.broadcast_to(cinv, (BR, L))


def _tc_out_body(p_ref, ci_ref, r_ref, wfc_ref, bfc_ref, o_ref):
  agg = p_ref[0] + p_ref[1]
  h = jax.nn.relu(agg * ci_ref[...] + r_ref[...])
  o_ref[...] = _mm(h, wfc_ref[...]) + bfc_ref[...]


def _row_spec(d):
  return pl.BlockSpec((BR, d), lambda i: (i, 0))


def _part_spec(d):
  return pl.BlockSpec((NC, BR, d), lambda i: (0, i, 0))


def _full_spec(a, b):
  return pl.BlockSpec((a, b), lambda i: (0, 0))


def kernel(x, edge_index, Wl1, bl1, Wr1, Wl2, bl2, Wr2, Wfc, bfc):
  ei = edge_index.astype(jnp.int32)
  pad = EPAD - E
  src = jnp.concatenate([ei[0], jnp.zeros((pad,), jnp.int32)])
  src = src.reshape(NCHUNK, CH)
  pad_dst = N + jnp.arange(pad, dtype=jnp.int32) % (N_SH - N)
  dst = jnp.concatenate([ei[1], pad_dst])
  dst = dst.reshape(NCHUNK, CH)

  grid = N // BR

  y1, r1 = pl.pallas_call(
      _tc_a_body,
      grid=(grid,),
      in_specs=[_row_spec(128), _full_spec(32, 128), _full_spec(32, 128),
                _full_spec(1, 32)],
      out_specs=[_row_spec(32), _row_spec(32)],
      out_shape=[jax.ShapeDtypeStruct((N, 32), jnp.float32)] * 2,
  )(x, Wl1, Wr1, bl1.reshape(1, 32))

  p1, c1 = _make_edge_agg(32, True)(y1, src, dst)

  y2, r2, cinv = pl.pallas_call(
      _tc_mid_body,
      grid=(grid,),
      in_specs=[_part_spec(32), _part_spec(L), _row_spec(32),
                _full_spec(16, 32), _full_spec(16, 32), _full_spec(1, 16)],
      out_specs=[_row_spec(16), _row_spec(16), _row_spec(L)],
      out_shape=[jax.ShapeDtypeStruct((N, 16), jnp.float32)] * 2
      + [jax.ShapeDtypeStruct((N, L), jnp.float32)],
  )(p1, c1, r1, Wl2, Wr2, bl2.reshape(1, 16))

  (p2,) = _make_edge_agg(16, False)(y2, src, dst)

  out = pl.pallas_call(
      _tc_out_body,
      grid=(grid,),
      in_specs=[_part_spec(16), _row_spec(L), _row_spec(16),
                _full_spec(2, 16), _full_spec(1, 2)],
      out_specs=_row_spec(2),
      out_shape=jax.ShapeDtypeStruct((N, 2), jnp.float32),
  )(p2, cinv, r2, Wfc, bfc.reshape(1, 2))
  return out


# speedup vs baseline: 1.6855x; 1.0463x over previous
import functools

import jax
import jax.numpy as jnp
from jax import lax
from jax.experimental import pallas as pl
from jax.experimental.pallas import tpu as pltpu
import jax.experimental.pallas.tpu_sc as plsc

N = 10000
E = 320000
NC, NS, L = 2, 16, 16
CH = 128
CPB = 8
NB0 = 10
NB1 = 10
NBMAX = max(NB0, NB1)
K0 = NB0 * CPB
K1 = NB1 * CPB
NCHUNK = NS * (K0 + K1)
EPAD = NCHUNK * CH
N_SH = NS * 640
BR = 1000


def _edge_agg_body(with_cnt, D, y_hbm, src_hbm, dst_hbm, *refs):
  if with_cnt:
    (out_hbm, cnt_hbm, srcw_v, dstw_v, rows_v, z_v, z16_v, ones_v,
     table, shared, shared_cnt) = refs[:11]
  else:
    (out_hbm, srcw_v, dstw_v, rows_v, z_v, table, shared) = refs[:7]
  sems = refs[-CPB:]
  c = lax.axis_index("c")
  s = lax.axis_index("s")

  base = jnp.where(c == 0, s * K0, NS * K0 + s * K1)
  t0 = s * 625

  @pl.when(c == 0)
  def _():
    pltpu.async_copy(src_hbm.at[pl.ds(base, K0)], srcw_v.at[pl.ds(0, K0)],
                     sems[5])
    pltpu.async_copy(dst_hbm.at[pl.ds(base, K0)], dstw_v.at[pl.ds(0, K0)],
                     sems[6])

  @pl.when(c != 0)
  def _():
    pltpu.async_copy(src_hbm.at[pl.ds(base, K1)], srcw_v.at[pl.ds(0, K1)],
                     sems[5])
    pltpu.async_copy(dst_hbm.at[pl.ds(base, K1)], dstw_v.at[pl.ds(0, K1)],
                     sems[6])

  for k in range(5):
    pltpu.async_copy(y_hbm.at[pl.ds(t0 + k * 125, 125)],
                     rows_v.at[k, pl.ds(0, 125)], sems[k])

  def zrow(i, _):
    for j in range(D // L):
      z_v[i, pl.ds(j * L, L)] = jnp.zeros((L,), jnp.float32)
    if with_cnt:
      z16_v[i, pl.ds(0, L)] = jnp.zeros((L,), jnp.float32)
      ones_v[i, pl.ds(0, L)] = jnp.ones((L,), jnp.float32)
    return 0
  lax.fori_loop(0, CH, zrow, 0, unroll=4)

  for k in range(5):
    pltpu.make_async_copy(y_hbm.at[pl.ds(t0, 125)],
                          rows_v.at[k, pl.ds(0, 125)], sems[k]).wait()
    pltpu.async_copy(rows_v.at[k, pl.ds(0, 125)],
                     table.at[pl.ds(t0 + k * 125, 125)], sems[k])
  r0 = s * 640
  def icopy(k, _):
    pltpu.sync_copy(z_v, shared.at[pl.ds(r0 + k * CH, CH)])
    if with_cnt:
      pltpu.sync_copy(z16_v, shared_cnt.at[pl.ds(r0 + k * CH, CH)])
    return 0
  lax.fori_loop(0, 640 // CH, icopy, 0)
  for k in range(5):
    pltpu.make_async_copy(rows_v.at[k, pl.ds(0, 125)],
                          table.at[pl.ds(t0, 125)], sems[k]).wait()

  @pl.when(c == 0)
  def _():
    pltpu.make_async_copy(src_hbm.at[pl.ds(0, K0)],
                          srcw_v.at[pl.ds(0, K0)], sems[5]).wait()
    pltpu.make_async_copy(dst_hbm.at[pl.ds(0, K0)],
                          dstw_v.at[pl.ds(0, K0)], sems[6]).wait()

  @pl.when(c != 0)
  def _():
    pltpu.make_async_copy(src_hbm.at[pl.ds(0, K1)],
                          srcw_v.at[pl.ds(0, K1)], sems[5]).wait()
    pltpu.make_async_copy(dst_hbm.at[pl.ds(0, K1)],
                          dstw_v.at[pl.ds(0, K1)], sems[6]).wait()

  plsc.subcore_barrier()

  for b in range(CPB):
    pltpu.async_copy(table.at[srcw_v.at[b]], rows_v.at[b], sems[b])

  def gwait(b):
    pltpu.make_async_copy(table.at[srcw_v.at[0]], rows_v.at[b],
                          sems[b]).wait()

  def swait(b):
    pltpu.make_async_copy(rows_v.at[b], shared.at[dstw_v.at[0]],
                          sems[b]).wait()
    if with_cnt:
      pltpu.make_async_copy(ones_v, shared_cnt.at[dstw_v.at[0]],
                            sems[b]).wait()

  def fire_scatter(b, j):
    pltpu.async_copy(rows_v.at[b], shared.at[dstw_v.at[j]], sems[b],
                     add=True)
    if with_cnt:
      pltpu.async_copy(ones_v, shared_cnt.at[dstw_v.at[j]], sems[b],
                       add=True)

  nb = jnp.where(c == 0, NB0, NB1)
  def pbody(g, _):
    for b in range(CPB):
      gwait(b)
      fire_scatter(b, g * CPB + b)
    for b in range(CPB):
      swait(b)
      pltpu.async_copy(table.at[srcw_v.at[(g + 1) * CPB + b]],
                       rows_v.at[b], sems[b])
    return 0
  lax.fori_loop(0, nb - 1, pbody, 0)
  for b in range(CPB):
    gwait(b)
    fire_scatter(b, (nb - 1) * CPB + b)
  for b in range(CPB):
    swait(b)
  plsc.subcore_barrier()

  for k in range(640 // CH):
    pltpu.sync_copy(shared.at[pl.ds(r0 + k * CH, CH)], rows_v.at[0])
    pltpu.sync_copy(rows_v.at[0], out_hbm.at[c, pl.ds(r0 + k * CH, CH)])
  if with_cnt:
    for k in range(640 // CH):
      pltpu.sync_copy(shared_cnt.at[pl.ds(r0 + k * CH, CH)], ones_v)
      pltpu.sync_copy(ones_v, cnt_hbm.at[c, pl.ds(r0 + k * CH, CH)])


def _make_edge_agg(D, with_cnt):
  mesh = plsc.VectorSubcoreMesh(core_axis_name="c", subcore_axis_name="s",
                                num_cores=NC, num_subcores=NS)
  out_type = [jax.ShapeDtypeStruct((NC, N_SH, D), jnp.float32)]
  scratch = [
      pltpu.VMEM((NBMAX * CPB, CH), jnp.int32),
      pltpu.VMEM((NBMAX * CPB, CH), jnp.int32),
      pltpu.VMEM((CPB, CH, D), jnp.float32),
      pltpu.VMEM((CH, D), jnp.float32),
  ]
  if with_cnt:
    out_type.append(jax.ShapeDtypeStruct((NC, N_SH, L), jnp.float32))
    scratch.append(pltpu.VMEM((CH, L), jnp.float32))
    scratch.append(pltpu.VMEM((CH, L), jnp.float32))
  scratch.append(pltpu.VMEM_SHARED((N, D), jnp.float32))
  scratch.append(pltpu.VMEM_SHARED((N_SH, D), jnp.float32))
  if with_cnt:
    scratch.append(pltpu.VMEM_SHARED((N_SH, L), jnp.float32))
  scratch.extend([pltpu.SemaphoreType.DMA] * CPB)
  return pl.kernel(
      functools.partial(_edge_agg_body, with_cnt, D),
      out_type=out_type, mesh=mesh, scratch_types=scratch,
      compiler_params=pltpu.CompilerParams(use_tc_tiling_on_sc=False),
      name=f"edge_agg_d{D}")


def _mm(x, w):
  return lax.dot_general(x, w, (((1,), (1,)), ((), ())),
                         preferred_element_type=jnp.float32)


def _tc_a_body(x_ref, wl_ref, wr_ref, bl_ref, y_ref, r_ref):
  xb = x_ref[...]
  y_ref[...] = _mm(xb, wl_ref[...])
  r_ref[...] = _mm(xb, wr_ref[...]) + bl_ref[...]


def _tc_mid_body(p_ref, c_ref, r_ref, wl_ref, wr_ref, bl_ref,
                 y_ref, r2_ref, ci_ref):
  agg = p_ref[0] + p_ref[1]
  cnt = (c_ref[0] + c_ref[1])[:, 0:1]
  cinv = 1.0 / jnp.maximum(cnt, 1.0)
  h = jax.nn.relu(agg * cinv + r_ref[...])
  y_ref[...] = _mm(h, wl_ref[...])
  r2_ref[...] = _mm(h, wr_ref[...]) + bl_ref[...]
  ci_ref[...] = jnp.broadcast_to(cinv, (BR, L))


def _tc_out_body(p_ref, ci_ref, r_ref, wfc_ref, bfc_ref, o_ref):
  agg = p_ref[0] + p_ref[1]
  h = jax.nn.relu(agg * ci_ref[...] + r_ref[...])
  o_ref[...] = _mm(h, wfc_ref[...]) + bfc_ref[...]


def _row_spec(d):
  return pl.BlockSpec((BR, d), lambda i: (i, 0))


def _part_spec(d):
  return pl.BlockSpec((NC, BR, d), lambda i: (0, i, 0))


def _full_spec(a, b):
  return pl.BlockSpec((a, b), lambda i: (0, 0))


def kernel(x, edge_index, Wl1, bl1, Wr1, Wl2, bl2, Wr2, Wfc, bfc):
  ei = edge_index.astype(jnp.int32)
  pad = EPAD - E
  src = jnp.concatenate([ei[0], jnp.zeros((pad,), jnp.int32)])
  src = src.reshape(NCHUNK, CH)
  pad_dst = N + (jnp.arange(pad, dtype=jnp.int32) & 127)
  dst = jnp.concatenate([ei[1], pad_dst])
  dst = dst.reshape(NCHUNK, CH)

  grid = N // BR

  y1, r1 = pl.pallas_call(
      _tc_a_body,
      grid=(grid,),
      in_specs=[_row_spec(128), _full_spec(32, 128), _full_spec(32, 128),
                _full_spec(1, 32)],
      out_specs=[_row_spec(32), _row_spec(32)],
      out_shape=[jax.ShapeDtypeStruct((N, 32), jnp.float32)] * 2,
  )(x, Wl1, Wr1, bl1.reshape(1, 32))

  p1, c1 = _make_edge_agg(32, True)(y1, src, dst)

  y2, r2, cinv = pl.pallas_call(
      _tc_mid_body,
      grid=(grid,),
      in_specs=[_part_spec(32), _part_spec(L), _row_spec(32),
                _full_spec(16, 32), _full_spec(16, 32), _full_spec(1, 16)],
      out_specs=[_row_spec(16), _row_spec(16), _row_spec(L)],
      out_shape=[jax.ShapeDtypeStruct((N, 16), jnp.float32)] * 2
      + [jax.ShapeDtypeStruct((N, L), jnp.float32)],
  )(p1, c1, r1, Wl2, Wr2, bl2.reshape(1, 16))

  (p2,) = _make_edge_agg(16, False)(y2, src, dst)

  out = pl.pallas_call(
      _tc_out_body,
      grid=(grid,),
      in_specs=[_part_spec(16), _row_spec(L), _row_spec(16),
                _full_spec(2, 16), _full_spec(1, 2)],
      out_specs=_row_spec(2),
      out_shape=jax.ShapeDtypeStruct((N, 2), jnp.float32),
  )(p2, cinv, r2, Wfc, bfc.reshape(1, 2))
  return out


# speedup vs baseline: 1.7846x; 1.0588x over previous
import functools

import jax
import jax.numpy as jnp
from jax import lax
from jax.experimental import pallas as pl
from jax.experimental.pallas import tpu as pltpu
import jax.experimental.pallas.tpu_sc as plsc

N = 10000
E = 320000
NC, NS, L = 2, 16, 16
CH = 128
CPB = 8
NB = 10
NCHUNK = E // CH
N_SH = NS * 640
BR = 1000


def _edge_agg_body(with_cnt, D, y_hbm, edges_hbm, *refs):
  if with_cnt:
    (out_hbm, cnt_hbm, srcw_v, dstw_v, rows_v, z_v, z16_v, ones_v,
     table, shared, shared_cnt) = refs[:11]
  else:
    (out_hbm, srcw_v, dstw_v, rows_v, z_v, table, shared) = refs[:7]
  sems = refs[-CPB:]
  c = lax.axis_index("c")
  s = lax.axis_index("s")
  w = c * NS + s

  for r in (NB * CPB - 2, NB * CPB - 1):
    for j in range(CH // L):
      srcw_v[r, pl.ds(j * L, L)] = jnp.zeros((L,), jnp.int32)
      dstw_v[r, pl.ds(j * L, L)] = (
          lax.iota(jnp.int32, L) + (N + j * L))

  base = 78 * w + jnp.minimum(w, 4)
  t0 = s * 625

  @pl.when(w < 4)
  def _():
    pltpu.async_copy(edges_hbm.at[0, pl.ds(base, 79)],
                     srcw_v.at[pl.ds(0, 79)], sems[5])
    pltpu.async_copy(edges_hbm.at[1, pl.ds(base, 79)],
                     dstw_v.at[pl.ds(0, 79)], sems[6])

  @pl.when(w >= 4)
  def _():
    pltpu.async_copy(edges_hbm.at[0, pl.ds(base, 78)],
                     srcw_v.at[pl.ds(0, 78)], sems[5])
    pltpu.async_copy(edges_hbm.at[1, pl.ds(base, 78)],
                     dstw_v.at[pl.ds(0, 78)], sems[6])

  for k in range(5):
    pltpu.async_copy(y_hbm.at[pl.ds(t0 + k * 125, 125)],
                     rows_v.at[k, pl.ds(0, 125)], sems[k])

  def zrow(i, _):
    for j in range(D // L):
      z_v[i, pl.ds(j * L, L)] = jnp.zeros((L,), jnp.float32)
    if with_cnt:
      z16_v[i, pl.ds(0, L)] = jnp.zeros((L,), jnp.float32)
      ones_v[i, pl.ds(0, L)] = jnp.ones((L,), jnp.float32)
    return 0
  lax.fori_loop(0, CH, zrow, 0, unroll=4)

  for k in range(5):
    pltpu.make_async_copy(y_hbm.at[pl.ds(t0, 125)],
                          rows_v.at[k, pl.ds(0, 125)], sems[k]).wait()
    pltpu.async_copy(rows_v.at[k, pl.ds(0, 125)],
                     table.at[pl.ds(t0 + k * 125, 125)], sems[k])
  r0 = s * 640
  def icopy(k, _):
    pltpu.sync_copy(z_v, shared.at[pl.ds(r0 + k * CH, CH)])
    if with_cnt:
      pltpu.sync_copy(z16_v, shared_cnt.at[pl.ds(r0 + k * CH, CH)])
    return 0
  lax.fori_loop(0, 640 // CH, icopy, 0)
  for k in range(5):
    pltpu.make_async_copy(rows_v.at[k, pl.ds(0, 125)],
                          table.at[pl.ds(t0, 125)], sems[k]).wait()

  @pl.when(w < 4)
  def _():
    pltpu.make_async_copy(edges_hbm.at[0, pl.ds(0, 79)],
                          srcw_v.at[pl.ds(0, 79)], sems[5]).wait()
    pltpu.make_async_copy(edges_hbm.at[1, pl.ds(0, 79)],
                          dstw_v.at[pl.ds(0, 79)], sems[6]).wait()

  @pl.when(w >= 4)
  def _():
    pltpu.make_async_copy(edges_hbm.at[0, pl.ds(0, 78)],
                          srcw_v.at[pl.ds(0, 78)], sems[5]).wait()
    pltpu.make_async_copy(edges_hbm.at[1, pl.ds(0, 78)],
                          dstw_v.at[pl.ds(0, 78)], sems[6]).wait()

  plsc.subcore_barrier()

  for b in range(CPB):
    pltpu.async_copy(table.at[srcw_v.at[b]], rows_v.at[b], sems[b])

  def gwait(b):
    pltpu.make_async_copy(table.at[srcw_v.at[0]], rows_v.at[b],
                          sems[b]).wait()

  def swait(b):
    pltpu.make_async_copy(rows_v.at[b], shared.at[dstw_v.at[0]],
                          sems[b]).wait()
    if with_cnt:
      pltpu.make_async_copy(ones_v, shared_cnt.at[dstw_v.at[0]],
                            sems[b]).wait()

  def fire_scatter(b, j):
    pltpu.async_copy(rows_v.at[b], shared.at[dstw_v.at[j]], sems[b],
                     add=True)
    if with_cnt:
      pltpu.async_copy(ones_v, shared_cnt.at[dstw_v.at[j]], sems[b],
                       add=True)

  def pbody(g, _):
    for b in range(CPB):
      gwait(b)
      fire_scatter(b, g * CPB + b)
    for b in range(CPB):
      swait(b)
      pltpu.async_copy(table.at[srcw_v.at[(g + 1) * CPB + b]],
                       rows_v.at[b], sems[b])
    return 0
  lax.fori_loop(0, NB - 1, pbody, 0)
  for b in range(CPB):
    gwait(b)
    fire_scatter(b, (NB - 1) * CPB + b)
  for b in range(CPB):
    swait(b)
  plsc.subcore_barrier()

  for k in range(640 // CH):
    pltpu.sync_copy(shared.at[pl.ds(r0 + k * CH, CH)], rows_v.at[0])
    pltpu.sync_copy(rows_v.at[0], out_hbm.at[c, pl.ds(r0 + k * CH, CH)])
  if with_cnt:
    for k in range(640 // CH):
      pltpu.sync_copy(shared_cnt.at[pl.ds(r0 + k * CH, CH)], ones_v)
      pltpu.sync_copy(ones_v, cnt_hbm.at[c, pl.ds(r0 + k * CH, CH)])


def _make_edge_agg(D, with_cnt):
  mesh = plsc.VectorSubcoreMesh(core_axis_name="c", subcore_axis_name="s",
                                num_cores=NC, num_subcores=NS)
  out_type = [jax.ShapeDtypeStruct((NC, N_SH, D), jnp.float32)]
  scratch = [
      pltpu.VMEM((NB * CPB, CH), jnp.int32),
      pltpu.VMEM((NB * CPB, CH), jnp.int32),
      pltpu.VMEM((CPB, CH, D), jnp.float32),
      pltpu.VMEM((CH, D), jnp.float32),
  ]
  if with_cnt:
    out_type.append(jax.ShapeDtypeStruct((NC, N_SH, L), jnp.float32))
    scratch.append(pltpu.VMEM((CH, L), jnp.float32))
    scratch.append(pltpu.VMEM((CH, L), jnp.float32))
  scratch.append(pltpu.VMEM_SHARED((N, D), jnp.float32))
  scratch.append(pltpu.VMEM_SHARED((N_SH, D), jnp.float32))
  if with_cnt:
    scratch.append(pltpu.VMEM_SHARED((N_SH, L), jnp.float32))
  scratch.extend([pltpu.SemaphoreType.DMA] * CPB)
  return pl.kernel(
      functools.partial(_edge_agg_body, with_cnt, D),
      out_type=out_type, mesh=mesh, scratch_types=scratch,
      compiler_params=pltpu.CompilerParams(use_tc_tiling_on_sc=False),
      name=f"edge_agg_d{D}")


def _mm(x, w):
  return lax.dot_general(x, w, (((1,), (1,)), ((), ())),
                         preferred_element_type=jnp.float32)


def _tc_a_body(x_ref, wl_ref, wr_ref, bl_ref, y_ref, r_ref):
  xb = x_ref[...]
  y_ref[...] = _mm(xb, wl_ref[...])
  r_ref[...] = _mm(xb, wr_ref[...]) + bl_ref[...]


def _tc_mid_body(p_ref, c_ref, r_ref, wl_ref, wr_ref, bl_ref,
                 y_ref, r2_ref, ci_ref):
  agg = p_ref[0] + p_ref[1]
  cnt = (c_ref[0] + c_ref[1])[:, 0:1]
  cinv = 1.0 / jnp.maximum(cnt, 1.0)
  h = jax.nn.relu(agg * cinv + r_ref[...])
  y_ref[...] = _mm(h, wl_ref[...])
  r2_ref[...] = _mm(h, wr_ref[...]) + bl_ref[...]
  ci_ref[...] = jnp.broadcast_to(cinv, (BR, L))


def _tc_out_body(p_ref, ci_ref, r_ref, wfc_ref, bfc_ref, o_ref):
  agg = p_ref[0] + p_ref[1]
  h = jax.nn.relu(agg * ci_ref[...] + r_ref[...])
  o_ref[...] = _mm(h, wfc_ref[...]) + bfc_ref[...]


def _row_spec(d):
  return pl.BlockSpec((BR, d), lambda i: (i, 0))


def _part_spec(d):
  return pl.BlockSpec((NC, BR, d), lambda i: (0, i, 0))


def _full_spec(a, b):
  return pl.BlockSpec((a, b), lambda i: (0, 0))


def kernel(x, edge_index, Wl1, bl1, Wr1, Wl2, bl2, Wr2, Wfc, bfc):
  edges = edge_index.astype(jnp.int32).reshape(2, NCHUNK, CH)

  grid = N // BR

  y1, r1 = pl.pallas_call(
      _tc_a_body,
      grid=(grid,),
      in_specs=[_row_spec(128), _full_spec(32, 128), _full_spec(32, 128),
                _full_spec(1, 32)],
      out_specs=[_row_spec(32), _row_spec(32)],
      out_shape=[jax.ShapeDtypeStruct((N, 32), jnp.float32)] * 2,
  )(x, Wl1, Wr1, bl1.reshape(1, 32))

  p1, c1 = _make_edge_agg(32, True)(y1, edges)

  y2, r2, cinv = pl.pallas_call(
      _tc_mid_body,
      grid=(grid,),
      in_specs=[_part_spec(32), _part_spec(L), _row_spec(32),
                _full_spec(16, 32), _full_spec(16, 32), _full_spec(1, 16)],
      out_specs=[_row_spec(16), _row_spec(16), _row_spec(L)],
      out_shape=[jax.ShapeDtypeStruct((N, 16), jnp.float32)] * 2
      + [jax.ShapeDtypeStruct((N, L), jnp.float32)],
  )(p1, c1, r1, Wl2, Wr2, bl2.reshape(1, 16))

  (p2,) = _make_edge_agg(16, False)(y2, edges)

  out = pl.pallas_call(
      _tc_out_body,
      grid=(grid,),
      in_specs=[_part_spec(16), _row_spec(L), _row_spec(16),
                _full_spec(2, 16), _full_spec(1, 2)],
      out_specs=_row_spec(2),
      out_shape=jax.ShapeDtypeStruct((N, 2), jnp.float32),
  )(p2, cinv, r2, Wfc, bfc.reshape(1, 2))
  return out
